# Initial kernel scaffold; baseline (speedup 1.0000x reference)
import jax
import jax.numpy as jnp
from jax.experimental import pallas as pl


def kernel(x, edge_index, weight, W1, b1, W2, b2, W3, b3):
    raise NotImplementedError("write your pallas kernel here")


# speedup vs baseline: 2.9664x; 2.9664x over previous
import functools

import jax
import jax.numpy as jnp
from jax import lax
from jax.experimental import pallas as pl
from jax.experimental.pallas import tpu as pltpu
from jax.experimental.pallas import tpu_sc as plsc

NN = 10000
NP = 10240
RPT = NP // 16
EE = 320000
GRP = 128
CH = 16
NCH = 10
G = CH * NCH
ET = G * GRP
EP = 16 * ET
DD = 128
HD = 64
CC = 10


def _rsqrt16(v):
    i = lax.bitcast_convert_type(v, jnp.int32)
    i = jnp.int32(0x5F3759DF) - (i >> 1)
    y = lax.bitcast_convert_type(i, jnp.float32)
    for _ in range(3):
        y = y * (1.5 - 0.5 * v * y * y)
    return jnp.where(v > 0.0, y, jnp.float32(0.0))


def _fill_zeros(ref, nrows, width):
    z = jnp.zeros((16,), jnp.float32)

    def body(i, _):
        for q in range(width // 16):
            ref[i, pl.ds(q * 16, 16)] = z
        return _

    lax.fori_loop(0, nrows, body, None, unroll=4)


def _zero_slice(zbuf, buf, base):
    for i in range(RPT // GRP):
        pltpu.sync_copy(zbuf, buf.at[pl.ds(base + i * GRP, GRP)])


def _hop(src_hbm, rowh_s, colh_s, nrmh_s, row_c, col_c, nrm_c, rows_v, acc):
    def chunk(t, _):
        pltpu.sync_copy(rowh_s.at[pl.ds(t * CH, CH)], row_c)
        pltpu.sync_copy(colh_s.at[pl.ds(t * CH, CH)], col_c)
        pltpu.sync_copy(nrmh_s.at[pl.ds(t * CH, CH)], nrm_c)

        def group(g, _):
            pltpu.sync_copy(src_hbm.at[row_c.at[g]], rows_v)

            def sub(u, _):
                nv = nrm_c[g, pl.ds(u * 16, 16)]
                for j in range(16):
                    w = nv[j]
                    r = u * 16 + j
                    for q in range(HD // 16):
                        sl = pl.ds(q * 16, 16)
                        rows_v[r, sl] = rows_v[r, sl] * w
                return _

            lax.fori_loop(0, GRP // 16, sub, None)
            pltpu.sync_copy(rows_v, acc.at[col_c.at[g]], add=True)
            return _

        lax.fori_loop(0, CH, group, None)
        return _

    lax.fori_loop(0, NCH, chunk, None)


def _run_hops(c, base, x2, outs, rowh_s, colh_s, nrmh_s,
              row_c, col_c, nrm_c, rows_v, zbuf, acc):
    srcs = (x2, outs[0], outs[1])
    for k in range(3):
        _hop(srcs[k].at[c], rowh_s, colh_s, nrmh_s,
             row_c, col_c, nrm_c, rows_v, acc)
        plsc.subcore_barrier()
        pltpu.sync_copy(acc.at[pl.ds(base, RPT)],
                        outs[k].at[c, pl.ds(base, RPT)])
        if k < 2:
            _zero_slice(zbuf, acc, base)
        plsc.subcore_barrier()


def _sc_layer1(x2, rowh, colh, ewh, normo, h1o, h2o, h3o,
               row_c, col_c, ew_c, nrm_c, dinv_v, rows_v, m16_v, dcmp_v, zbuf,
               deg_s, dinv_s, acc):
    c = lax.axis_index("c")
    s = lax.axis_index("s")
    base = s * RPT
    rowh_s, colh_s, ewh_s = rowh.at[s], colh.at[s], ewh.at[s]

    _fill_zeros(zbuf, GRP, HD)
    _fill_zeros(m16_v, RPT, 16)
    pltpu.sync_copy(m16_v, deg_s.at[pl.ds(base, RPT)])
    _zero_slice(zbuf, acc, base)
    plsc.subcore_barrier()

    def dchunk(t, _):
        pltpu.sync_copy(colh_s.at[pl.ds(t * CH, CH)], col_c)
        pltpu.sync_copy(ewh_s.at[pl.ds(t * CH, CH)], ew_c)

        def dg(g, _):
            def dsub(u, _):
                wv = ew_c[g, pl.ds(u * 16, 16)]
                for j in range(16):
                    m16_v[u * 16 + j, :] = jnp.full((16,), wv[j], jnp.float32)
                return _

            lax.fori_loop(0, GRP // 16, dsub, None)
            pltpu.sync_copy(m16_v.at[pl.ds(0, GRP)],
                            deg_s.at[col_c.at[g]], add=True)
            return _

        lax.fori_loop(0, CH, dg, None)
        return _

    lax.fori_loop(0, NCH, dchunk, None)
    plsc.subcore_barrier()

    pltpu.sync_copy(deg_s.at[pl.ds(base, RPT)], m16_v)
    lanes0 = jnp.zeros((16,), jnp.int32)

    def di(i, _):
        rows16 = i * 16 + lax.iota(jnp.int32, 16)
        d = plsc.load_gather(m16_v, [rows16, lanes0])
        dcmp_v[pl.ds(i * 16, 16)] = _rsqrt16(d)
        return _

    lax.fori_loop(0, RPT // 16, di, None)
    pltpu.sync_copy(dcmp_v, dinv_s.at[pl.ds(base, RPT)])
    plsc.subcore_barrier()
    pltpu.sync_copy(dinv_s, dinv_v)

    def nchunk(t, _):
        pltpu.sync_copy(rowh_s.at[pl.ds(t * CH, CH)], row_c)
        pltpu.sync_copy(colh_s.at[pl.ds(t * CH, CH)], col_c)
        pltpu.sync_copy(ewh_s.at[pl.ds(t * CH, CH)], ew_c)

        def ng(g, _):
            def nsub(q, _):
                sl = pl.ds(q * 16, 16)
                a = plsc.load_gather(dinv_v, [row_c[g, sl]])
                b = plsc.load_gather(dinv_v, [col_c[g, sl]])
                nrm_c[g, sl] = a * b * ew_c[g, sl]
                return _

            lax.fori_loop(0, GRP // 16, nsub, None)
            return _

        lax.fori_loop(0, CH, ng, None)

        @pl.when(c == 0)
        def _():
            pltpu.sync_copy(nrm_c, normo.at[s, pl.ds(t * CH, CH)])

        return _

    lax.fori_loop(0, NCH, nchunk, None)
    plsc.subcore_barrier()

    _run_hops(c, base, x2, (h1o, h2o, h3o), rowh_s, colh_s, normo.at[s],
              row_c, col_c, nrm_c, rows_v, zbuf, acc)


def _sc_prop(yp, rowh, colh, normh, h1o, h2o, h3o,
             row_c, col_c, nrm_c, rows_v, zbuf, acc):
    c = lax.axis_index("c")
    s = lax.axis_index("s")
    base = s * RPT

    _fill_zeros(zbuf, GRP, HD)
    _zero_slice(zbuf, acc, base)
    plsc.subcore_barrier()

    _run_hops(c, base, yp, (h1o, h2o, h3o), rowh.at[s], colh.at[s], normh.at[s],
              row_c, col_c, nrm_c, rows_v, zbuf, acc)


_MESH = plsc.VectorSubcoreMesh(core_axis_name="c", subcore_axis_name="s")

_F32 = jnp.float32
_HSHAPE = jax.ShapeDtypeStruct((2, NP, HD), _F32)

_SC_PARAMS = pltpu.CompilerParams(use_tc_tiling_on_sc=False,
                                  needs_layout_passes=False)

_layer1_call = pl.kernel(
    _sc_layer1,
    out_type=(jax.ShapeDtypeStruct((16, G, GRP), _F32), _HSHAPE, _HSHAPE, _HSHAPE),
    mesh=_MESH,
    compiler_params=_SC_PARAMS,
    scratch_types=[
        pltpu.VMEM((CH, GRP), jnp.int32),
        pltpu.VMEM((CH, GRP), jnp.int32),
        pltpu.VMEM((CH, GRP), _F32),
        pltpu.VMEM((CH, GRP), _F32),
        pltpu.VMEM((NP,), _F32),
        pltpu.VMEM((GRP, HD), _F32),
        pltpu.VMEM((RPT, 16), _F32),
        pltpu.VMEM((RPT,), _F32),
        pltpu.VMEM((GRP, HD), _F32),
        pltpu.VMEM_SHARED((NP, 16), _F32),
        pltpu.VMEM_SHARED((NP,), _F32),
        pltpu.VMEM_SHARED((NP, HD), _F32),
    ],
)

_prop_call = pl.kernel(
    _sc_prop,
    out_type=(_HSHAPE, _HSHAPE, _HSHAPE),
    mesh=_MESH,
    compiler_params=_SC_PARAMS,
    scratch_types=[
        pltpu.VMEM((CH, GRP), jnp.int32),
        pltpu.VMEM((CH, GRP), jnp.int32),
        pltpu.VMEM((CH, GRP), _F32),
        pltpu.VMEM((GRP, HD), _F32),
        pltpu.VMEM((GRP, HD), _F32),
        pltpu.VMEM_SHARED((NP, HD), _F32),
    ],
)


def _tc_body(x_ref, h1_ref, h2_ref, h3_ref, w_ref, b_ref, o_ref, *, act, split):
    def cat(r):
        return jnp.concatenate([r[0], r[1]], axis=1)

    acc = jnp.dot(cat(x_ref), w_ref[0], preferred_element_type=_F32)
    acc = acc + jnp.dot(cat(h1_ref), w_ref[1], preferred_element_type=_F32)
    acc = acc + jnp.dot(cat(h2_ref), w_ref[2], preferred_element_type=_F32)
    acc = acc + jnp.dot(cat(h3_ref), w_ref[3], preferred_element_type=_F32)
    acc = acc + b_ref[...]
    if act:
        acc = jnp.where(acc > 0.0, acc, jnp.exp(jnp.minimum(acc, 0.0)) - 1.0)
    if split:
        o_ref[0] = acc[:, :HD]
        o_ref[1] = acc[:, HD:]
    else:
        o_ref[...] = acc


def _dense(x2, h1, h2, h3, w, b, act, split):
    bn = 512
    body = functools.partial(_tc_body, act=act, split=split)
    if split:
        out_shape = jax.ShapeDtypeStruct((2, NP, HD), _F32)
        out_spec = pl.BlockSpec((2, bn, HD), lambda i: (0, i, 0))
    else:
        out_shape = jax.ShapeDtypeStruct((NP, DD), _F32)
        out_spec = pl.BlockSpec((bn, DD), lambda i: (i, 0))
    return pl.pallas_call(
        body,
        grid=(NP // bn,),
        in_specs=[pl.BlockSpec((2, bn, HD), lambda i: (0, i, 0))] * 4
        + [pl.BlockSpec((4, DD, DD), lambda i: (0, 0, 0)),
           pl.BlockSpec((1, DD), lambda i: (0, 0))],
        out_specs=out_spec,
        out_shape=out_shape,
    )(x2, h1, h2, h3, w, b)


def kernel(x, edge_index, weight, W1, b1, W2, b2, W3, b3):
    row, col = edge_index[0], edge_index[1]
    pad_e = EP - EE
    x2 = jnp.pad(x, ((0, NP - NN), (0, 0))).reshape(NP, 2, HD).transpose(1, 0, 2)
    rowp = jnp.concatenate(
        [row, jnp.full((pad_e,), NP - 1, jnp.int32)]).reshape(16, G, GRP)
    colp = jnp.concatenate(
        [col, jnp.full((pad_e,), NP - 1, jnp.int32)]).reshape(16, G, GRP)
    ewp = jnp.concatenate(
        [weight, jnp.zeros((pad_e,), _F32)]).reshape(16, G, GRP)

    norm, h1, h2, h3 = _layer1_call(x2, rowp, colp, ewp)
    y = _dense(x2, h1, h2, h3, W1, b1.reshape(1, DD), act=True, split=True)
    h1, h2, h3 = _prop_call(y, rowp, colp, norm)
    y = _dense(y, h1, h2, h3, W2, b2.reshape(1, DD), act=True, split=True)
    h1, h2, h3 = _prop_call(y, rowp, colp, norm)
    w3p = jnp.pad(W3, ((0, 0), (0, 0), (0, DD - CC)))
    b3p = jnp.pad(b3, (0, DD - CC)).reshape(1, DD)
    y = _dense(y, h1, h2, h3, w3p, b3p, act=False, split=False)
    return y[:NN, :CC]


# speedup vs baseline: 4.0784x; 1.3749x over previous
import functools

import jax
import jax.numpy as jnp
from jax import lax
from jax.experimental import pallas as pl
from jax.experimental.pallas import tpu as pltpu
from jax.experimental.pallas import tpu_sc as plsc

NN = 10000
NP = 10240
RPT = NP // 16
EE = 320000
GRP = 128
CH = 16
NCH = 10
G = CH * NCH
ET = G * GRP
EP = 16 * ET
DD = 128
HD = 64
CC = 10


def _rsqrt16(v):
    i = lax.bitcast_convert_type(v, jnp.int32)
    i = jnp.int32(0x5F3759DF) - (i >> 1)
    y = lax.bitcast_convert_type(i, jnp.float32)
    for _ in range(3):
        y = y * (1.5 - 0.5 * v * y * y)
    return jnp.where(v > 0.0, y, jnp.float32(0.0))


def _fill_zeros(ref, nrows, width):
    z = jnp.zeros((16,), jnp.float32)

    def body(i, _):
        for q in range(width // 16):
            ref[i, pl.ds(q * 16, 16)] = z
        return _

    lax.fori_loop(0, nrows, body, None, unroll=4)


def _zero_slice(zbuf, buf, base):
    for i in range(RPT // GRP):
        pltpu.sync_copy(zbuf, buf.at[pl.ds(base + i * GRP, GRP)])


def _hop(src_hbm, rowh_s, colh_s, nrmh_s, row_c, col_c, nrm_c,
         rows_v, rows_w, gsem, acc):
    bufs = (rows_v, rows_w)

    def scale_scatter(g, buf):
        def sub(u, _):
            nv = nrm_c[g, pl.ds(u * 16, 16)]
            for j in range(16):
                w = nv[j]
                r = u * 16 + j
                for q in range(HD // 16):
                    sl = pl.ds(q * 16, 16)
                    buf[r, sl] = buf[r, sl] * w
            return _

        lax.fori_loop(0, GRP // 16, sub, None)
        pltpu.sync_copy(buf, acc.at[col_c.at[g]], add=True)

    def chunk(t, _):
        pltpu.sync_copy(rowh_s.at[pl.ds(t * CH, CH)], row_c)
        pltpu.sync_copy(colh_s.at[pl.ds(t * CH, CH)], col_c)
        pltpu.sync_copy(nrmh_s.at[pl.ds(t * CH, CH)], nrm_c)
        pltpu.async_copy(src_hbm.at[row_c.at[0]], bufs[0], gsem)

        def pair(h, _):
            for b in range(2):
                g = h * 2 + b
                pltpu.make_async_copy(
                    src_hbm.at[row_c.at[g]], bufs[b], gsem).wait()

                @pl.when(g < CH - 1)
                def _():
                    pltpu.async_copy(
                        src_hbm.at[row_c.at[g + 1]], bufs[1 - b], gsem)

                scale_scatter(g, bufs[b])
            return _

        lax.fori_loop(0, CH // 2, pair, None)
        return _

    lax.fori_loop(0, NCH, chunk, None)


def _run_hops(c, base, x2, outs, rowh_s, colh_s, nrmh_s,
              row_c, col_c, nrm_c, rows_v, rows_w, gsem, zbuf, acc):
    srcs = (x2, outs[0], outs[1])
    for k in range(3):
        _hop(srcs[k].at[c], rowh_s, colh_s, nrmh_s,
             row_c, col_c, nrm_c, rows_v, rows_w, gsem, acc)
        plsc.subcore_barrier()
        pltpu.sync_copy(acc.at[pl.ds(base, RPT)],
                        outs[k].at[c, pl.ds(base, RPT)])
        if k < 2:
            _zero_slice(zbuf, acc, base)
        plsc.subcore_barrier()


def _sc_layer1(x2, rowh, colh, ewh, normo, h1o, h2o, h3o,
               row_c, col_c, ew_c, nrm_c, dinv_v, rows_v, rows_w, m16_v,
               dcmp_v, zbuf, gsem, deg_s, dinv_s, acc):
    c = lax.axis_index("c")
    s = lax.axis_index("s")
    base = s * RPT
    rowh_s, colh_s, ewh_s = rowh.at[s], colh.at[s], ewh.at[s]

    _fill_zeros(zbuf, GRP, HD)
    _fill_zeros(m16_v, RPT, 16)
    pltpu.sync_copy(m16_v, deg_s.at[pl.ds(base, RPT)])
    _zero_slice(zbuf, acc, base)
    plsc.subcore_barrier()

    def dchunk(t, _):
        pltpu.sync_copy(colh_s.at[pl.ds(t * CH, CH)], col_c)
        pltpu.sync_copy(ewh_s.at[pl.ds(t * CH, CH)], ew_c)

        def dg(g, _):
            def dsub(u, _):
                wv = ew_c[g, pl.ds(u * 16, 16)]
                for j in range(16):
                    m16_v[u * 16 + j, :] = jnp.full((16,), wv[j], jnp.float32)
                return _

            lax.fori_loop(0, GRP // 16, dsub, None)
            pltpu.sync_copy(m16_v.at[pl.ds(0, GRP)],
                            deg_s.at[col_c.at[g]], add=True)
            return _

        lax.fori_loop(0, CH, dg, None)
        return _

    lax.fori_loop(0, NCH, dchunk, None)
    plsc.subcore_barrier()

    pltpu.sync_copy(deg_s.at[pl.ds(base, RPT)], m16_v)
    lanes0 = jnp.zeros((16,), jnp.int32)

    def di(i, _):
        rows16 = i * 16 + lax.iota(jnp.int32, 16)
        d = plsc.load_gather(m16_v, [rows16, lanes0])
        dcmp_v[pl.ds(i * 16, 16)] = _rsqrt16(d)
        return _

    lax.fori_loop(0, RPT // 16, di, None)
    pltpu.sync_copy(dcmp_v, dinv_s.at[pl.ds(base, RPT)])
    plsc.subcore_barrier()
    pltpu.sync_copy(dinv_s, dinv_v)

    def nchunk(t, _):
        pltpu.sync_copy(rowh_s.at[pl.ds(t * CH, CH)], row_c)
        pltpu.sync_copy(colh_s.at[pl.ds(t * CH, CH)], col_c)
        pltpu.sync_copy(ewh_s.at[pl.ds(t * CH, CH)], ew_c)

        def ng(g, _):
            def nsub(q, _):
                sl = pl.ds(q * 16, 16)
                a = plsc.load_gather(dinv_v, [row_c[g, sl]])
                b = plsc.load_gather(dinv_v, [col_c[g, sl]])
                nrm_c[g, sl] = a * b * ew_c[g, sl]
                return _

            lax.fori_loop(0, GRP // 16, nsub, None)
            return _

        lax.fori_loop(0, CH, ng, None)

        @pl.when(c == 0)
        def _():
            pltpu.sync_copy(nrm_c, normo.at[s, pl.ds(t * CH, CH)])

        return _

    lax.fori_loop(0, NCH, nchunk, None)
    plsc.subcore_barrier()

    _run_hops(c, base, x2, (h1o, h2o, h3o), rowh_s, colh_s, normo.at[s],
              row_c, col_c, nrm_c, rows_v, rows_w, gsem, zbuf, acc)


def _sc_prop(yp, rowh, colh, normh, h1o, h2o, h3o,
             row_c, col_c, nrm_c, rows_v, rows_w, zbuf, gsem, acc):
    c = lax.axis_index("c")
    s = lax.axis_index("s")
    base = s * RPT

    _fill_zeros(zbuf, GRP, HD)
    _zero_slice(zbuf, acc, base)
    plsc.subcore_barrier()

    _run_hops(c, base, yp, (h1o, h2o, h3o), rowh.at[s], colh.at[s], normh.at[s],
              row_c, col_c, nrm_c, rows_v, rows_w, gsem, zbuf, acc)


_MESH = plsc.VectorSubcoreMesh(core_axis_name="c", subcore_axis_name="s")

_F32 = jnp.float32
_HSHAPE = jax.ShapeDtypeStruct((2, NP, HD), _F32)

_SC_PARAMS = pltpu.CompilerParams(use_tc_tiling_on_sc=False,
                                  needs_layout_passes=False)

_layer1_call = pl.kernel(
    _sc_layer1,
    out_type=(jax.ShapeDtypeStruct((16, G, GRP), _F32), _HSHAPE, _HSHAPE, _HSHAPE),
    mesh=_MESH,
    compiler_params=_SC_PARAMS,
    scratch_types=[
        pltpu.VMEM((CH, GRP), jnp.int32),
        pltpu.VMEM((CH, GRP), jnp.int32),
        pltpu.VMEM((CH, GRP), _F32),
        pltpu.VMEM((CH, GRP), _F32),
        pltpu.VMEM((NP,), _F32),
        pltpu.VMEM((GRP, HD), _F32),
        pltpu.VMEM((GRP, HD), _F32),
        pltpu.VMEM((RPT, 16), _F32),
        pltpu.VMEM((RPT,), _F32),
        pltpu.VMEM((GRP, HD), _F32),
        pltpu.SemaphoreType.DMA,
        pltpu.VMEM_SHARED((NP, 16), _F32),
        pltpu.VMEM_SHARED((NP,), _F32),
        pltpu.VMEM_SHARED((NP, HD), _F32),
    ],
)

_prop_call = pl.kernel(
    _sc_prop,
    out_type=(_HSHAPE, _HSHAPE, _HSHAPE),
    mesh=_MESH,
    compiler_params=_SC_PARAMS,
    scratch_types=[
        pltpu.VMEM((CH, GRP), jnp.int32),
        pltpu.VMEM((CH, GRP), jnp.int32),
        pltpu.VMEM((CH, GRP), _F32),
        pltpu.VMEM((GRP, HD), _F32),
        pltpu.VMEM((GRP, HD), _F32),
        pltpu.VMEM((GRP, HD), _F32),
        pltpu.SemaphoreType.DMA,
        pltpu.VMEM_SHARED((NP, HD), _F32),
    ],
)


def _tc_body(x_ref, h1_ref, h2_ref, h3_ref, w_ref, b_ref, o_ref, *, act, split):
    def cat(r):
        return jnp.concatenate([r[0], r[1]], axis=1)

    acc = jnp.dot(cat(x_ref), w_ref[0], preferred_element_type=_F32)
    acc = acc + jnp.dot(cat(h1_ref), w_ref[1], preferred_element_type=_F32)
    acc = acc + jnp.dot(cat(h2_ref), w_ref[2], preferred_element_type=_F32)
    acc = acc + jnp.dot(cat(h3_ref), w_ref[3], preferred_element_type=_F32)
    acc = acc + b_ref[...]
    if act:
        acc = jnp.where(acc > 0.0, acc, jnp.exp(jnp.minimum(acc, 0.0)) - 1.0)
    if split:
        o_ref[0] = acc[:, :HD]
        o_ref[1] = acc[:, HD:]
    else:
        o_ref[...] = acc


def _dense(x2, h1, h2, h3, w, b, act, split):
    bn = 512
    body = functools.partial(_tc_body, act=act, split=split)
    if split:
        out_shape = jax.ShapeDtypeStruct((2, NP, HD), _F32)
        out_spec = pl.BlockSpec((2, bn, HD), lambda i: (0, i, 0))
    else:
        out_shape = jax.ShapeDtypeStruct((NP, DD), _F32)
        out_spec = pl.BlockSpec((bn, DD), lambda i: (i, 0))
    return pl.pallas_call(
        body,
        grid=(NP // bn,),
        in_specs=[pl.BlockSpec((2, bn, HD), lambda i: (0, i, 0))] * 4
        + [pl.BlockSpec((4, DD, DD), lambda i: (0, 0, 0)),
           pl.BlockSpec((1, DD), lambda i: (0, 0))],
        out_specs=out_spec,
        out_shape=out_shape,
    )(x2, h1, h2, h3, w, b)


def kernel(x, edge_index, weight, W1, b1, W2, b2, W3, b3):
    row, col = edge_index[0], edge_index[1]
    pad_e = EP - EE
    x2 = jnp.pad(x, ((0, NP - NN), (0, 0))).reshape(NP, 2, HD).transpose(1, 0, 2)
    rowp = jnp.concatenate(
        [row, jnp.full((pad_e,), NP - 1, jnp.int32)]).reshape(16, G, GRP)
    colp = jnp.concatenate(
        [col, jnp.full((pad_e,), NP - 1, jnp.int32)]).reshape(16, G, GRP)
    ewp = jnp.concatenate(
        [weight, jnp.zeros((pad_e,), _F32)]).reshape(16, G, GRP)

    norm, h1, h2, h3 = _layer1_call(x2, rowp, colp, ewp)
    y = _dense(x2, h1, h2, h3, W1, b1.reshape(1, DD), act=True, split=True)
    h1, h2, h3 = _prop_call(y, rowp, colp, norm)
    y = _dense(y, h1, h2, h3, W2, b2.reshape(1, DD), act=True, split=True)
    h1, h2, h3 = _prop_call(y, rowp, colp, norm)
    w3p = jnp.pad(W3, ((0, 0), (0, 0), (0, DD - CC)))
    b3p = jnp.pad(b3, (0, DD - CC)).reshape(1, DD)
    y = _dense(y, h1, h2, h3, w3p, b3p, act=False, split=False)
    return y[:NN, :CC]


# speedup vs baseline: 4.3013x; 1.0547x over previous
import functools

import jax
import jax.numpy as jnp
from jax import lax
from jax.experimental import pallas as pl
from jax.experimental.pallas import tpu as pltpu
from jax.experimental.pallas import tpu_sc as plsc

NN = 10000
NP = 10240
RPT = NP // 16
EE = 320000
GRP = 128
CH = 16
NCH = 10
G = CH * NCH
ET = G * GRP
EP = 16 * ET
DD = 128
HD = 64
CC = 10


def _rsqrt16(v):
    i = lax.bitcast_convert_type(v, jnp.int32)
    i = jnp.int32(0x5F3759DF) - (i >> 1)
    y = lax.bitcast_convert_type(i, jnp.float32)
    for _ in range(3):
        y = y * (1.5 - 0.5 * v * y * y)
    return jnp.where(v > 0.0, y, jnp.float32(0.0))


def _fill_zeros(ref, nrows, width):
    z = jnp.zeros((16,), jnp.float32)

    def body(i, _):
        for q in range(width // 16):
            ref[i, pl.ds(q * 16, 16)] = z
        return _

    lax.fori_loop(0, nrows, body, None, unroll=4)


def _zero_slice(zbuf, buf, base):
    for i in range(RPT // GRP):
        pltpu.sync_copy(zbuf, buf.at[pl.ds(base + i * GRP, GRP)])


def _scale_scatter(buf, col_f, tg, nrm_ref, g, acc):
    def sub(u, _):
        nv = nrm_ref[g, pl.ds(u * 16, 16)]
        for j in range(16):
            w = nv[j]
            r = u * 16 + j
            for q in range(HD // 16):
                sl = pl.ds(q * 16, 16)
                buf[r, sl] = buf[r, sl] * w
        return _

    lax.fori_loop(0, GRP // 16, sub, None)
    pltpu.sync_copy(buf, acc.at[col_f.at[tg]], add=True)


def _hop_res(src_hbm, row_f, col_f, nrm_f, rows_v, rows_w, gsem, acc):
    bufs = (rows_v, rows_w)
    pltpu.async_copy(src_hbm.at[row_f.at[0]], bufs[0], gsem)

    def pair(h, _):
        for b in range(2):
            g = h * 2 + b
            pltpu.make_async_copy(
                src_hbm.at[row_f.at[g]], bufs[b], gsem).wait()

            @pl.when(g < G - 1)
            def _():
                pltpu.async_copy(
                    src_hbm.at[row_f.at[g + 1]], bufs[1 - b], gsem)

            _scale_scatter(bufs[b], col_f, g, nrm_f, g, acc)
        return _

    lax.fori_loop(0, G // 2, pair, None)


def _hop_chk(src_hbm, row_f, col_f, nrmh_s, nrm_c, rows_v, rows_w, gsem, acc):
    bufs = (rows_v, rows_w)

    def chunk(t, _):
        pltpu.sync_copy(nrmh_s.at[pl.ds(t * CH, CH)], nrm_c)
        pltpu.async_copy(src_hbm.at[row_f.at[t * CH]], bufs[0], gsem)

        def pair(h, _):
            for b in range(2):
                g = h * 2 + b
                tg = t * CH + g
                pltpu.make_async_copy(
                    src_hbm.at[row_f.at[tg]], bufs[b], gsem).wait()

                @pl.when(g < CH - 1)
                def _():
                    pltpu.async_copy(
                        src_hbm.at[row_f.at[tg + 1]], bufs[1 - b], gsem)

                _scale_scatter(bufs[b], col_f, tg, nrm_c, g, acc)
            return _

        lax.fori_loop(0, CH // 2, pair, None)
        return _

    lax.fori_loop(0, NCH, chunk, None)


def _hop_epilogue(c, base, out, k, zbuf, acc):
    plsc.subcore_barrier()
    pltpu.sync_copy(acc.at[pl.ds(base, RPT)], out.at[c, pl.ds(base, RPT)])
    if k < 2:
        _zero_slice(zbuf, acc, base)
    plsc.subcore_barrier()


def _sc_layer1(x2, rowh, colh, ewh, normo, h1o, h2o, h3o,
               row_f, col_f, ew_c, nrm_c, dinv2, idx_v, rows_v, rows_w,
               zbuf, gsem, deg_s, acc):
    c = lax.axis_index("c")
    s = lax.axis_index("s")
    base = s * RPT
    ewh_s = ewh.at[s]

    pltpu.sync_copy(rowh.at[s], row_f)
    pltpu.sync_copy(colh.at[s], col_f)
    _fill_zeros(zbuf, GRP, HD)
    _fill_zeros(dinv2, NP // 16, 16)
    iota16 = lax.iota(jnp.int32, 16)

    def zi(i, _):
        idx_v[pl.ds(i * 16, 16)] = i * 16 + iota16
        return _

    lax.fori_loop(0, NP // 256, zi, None, unroll=4)

    @pl.when(s == 0)
    def _():
        pltpu.sync_copy(dinv2, deg_s)

    _zero_slice(zbuf, acc, base)
    plsc.subcore_barrier()

    def dchunk(t, _):
        pltpu.sync_copy(ewh_s.at[pl.ds(t * CH, CH)], ew_c)

        def dg(g, _):
            tg = t * CH + g

            def dsub(u, _):
                sl = pl.ds(u * 16, 16)
                cv = col_f[tg, sl]
                plsc.addupdate_scatter(
                    dinv2, [cv >> 4, cv & 15], ew_c[g, sl])
                return _

            lax.fori_loop(0, GRP // 16, dsub, None)
            return _

        lax.fori_loop(0, CH, dg, None)
        return _

    lax.fori_loop(0, NCH, dchunk, None)
    pltpu.sync_copy(dinv2, deg_s.at[idx_v], add=True)
    plsc.subcore_barrier()

    pltpu.sync_copy(deg_s, dinv2)

    def di(i, _):
        dinv2[i, :] = _rsqrt16(dinv2[i, :])
        return _

    lax.fori_loop(0, NP // 16, di, None)

    def nchunk(t, _):
        pltpu.sync_copy(ewh_s.at[pl.ds(t * CH, CH)], ew_c)

        def ng(g, _):
            tg = t * CH + g

            def nsub(q, _):
                sl = pl.ds(q * 16, 16)
                rv = row_f[tg, sl]
                cv = col_f[tg, sl]
                a = plsc.load_gather(dinv2, [rv >> 4, rv & 15])
                b = plsc.load_gather(dinv2, [cv >> 4, cv & 15])
                nrm_c[g, sl] = a * b * ew_c[g, sl]
                return _

            lax.fori_loop(0, GRP // 16, nsub, None)
            return _

        lax.fori_loop(0, CH, ng, None)

        @pl.when(c == 0)
        def _():
            pltpu.sync_copy(nrm_c, normo.at[s, pl.ds(t * CH, CH)])

        return _

    lax.fori_loop(0, NCH, nchunk, None)
    plsc.subcore_barrier()

    srcs = (x2, h1o, h2o)
    outs = (h1o, h2o, h3o)
    for k in range(3):
        _hop_chk(srcs[k].at[c], row_f, col_f, normo.at[s], nrm_c,
                 rows_v, rows_w, gsem, acc)
        _hop_epilogue(c, base, outs[k], k, zbuf, acc)


def _sc_prop(yp, rowh, colh, normh, h1o, h2o, h3o,
             row_f, col_f, nrm_f, rows_v, rows_w, zbuf, gsem, acc):
    c = lax.axis_index("c")
    s = lax.axis_index("s")
    base = s * RPT

    pltpu.sync_copy(rowh.at[s], row_f)
    pltpu.sync_copy(colh.at[s], col_f)
    pltpu.sync_copy(normh.at[s], nrm_f)
    _fill_zeros(zbuf, GRP, HD)
    _zero_slice(zbuf, acc, base)
    plsc.subcore_barrier()

    srcs = (yp, h1o, h2o)
    outs = (h1o, h2o, h3o)
    for k in range(3):
        _hop_res(srcs[k].at[c], row_f, col_f, nrm_f,
                 rows_v, rows_w, gsem, acc)
        _hop_epilogue(c, base, outs[k], k, zbuf, acc)


_MESH = plsc.VectorSubcoreMesh(core_axis_name="c", subcore_axis_name="s")

_F32 = jnp.float32
_HSHAPE = jax.ShapeDtypeStruct((2, NP, HD), _F32)

_SC_PARAMS = pltpu.CompilerParams(use_tc_tiling_on_sc=False,
                                  needs_layout_passes=False)

_layer1_call = pl.kernel(
    _sc_layer1,
    out_type=(jax.ShapeDtypeStruct((16, G, GRP), _F32), _HSHAPE, _HSHAPE, _HSHAPE),
    mesh=_MESH,
    compiler_params=_SC_PARAMS,
    scratch_types=[
        pltpu.VMEM((G, GRP), jnp.int32),
        pltpu.VMEM((G, GRP), jnp.int32),
        pltpu.VMEM((CH, GRP), _F32),
        pltpu.VMEM((CH, GRP), _F32),
        pltpu.VMEM((NP // 16, 16), _F32),
        pltpu.VMEM((NP // 16,), jnp.int32),
        pltpu.VMEM((GRP, HD), _F32),
        pltpu.VMEM((GRP, HD), _F32),
        pltpu.VMEM((GRP, HD), _F32),
        pltpu.SemaphoreType.DMA,
        pltpu.VMEM_SHARED((NP // 16, 16), _F32),
        pltpu.VMEM_SHARED((NP, HD), _F32),
    ],
)

_prop_call = pl.kernel(
    _sc_prop,
    out_type=(_HSHAPE, _HSHAPE, _HSHAPE),
    mesh=_MESH,
    compiler_params=_SC_PARAMS,
    scratch_types=[
        pltpu.VMEM((G, GRP), jnp.int32),
        pltpu.VMEM((G, GRP), jnp.int32),
        pltpu.VMEM((G, GRP), _F32),
        pltpu.VMEM((GRP, HD), _F32),
        pltpu.VMEM((GRP, HD), _F32),
        pltpu.VMEM((GRP, HD), _F32),
        pltpu.SemaphoreType.DMA,
        pltpu.VMEM_SHARED((NP, HD), _F32),
    ],
)


def _tc_body(x_ref, h1_ref, h2_ref, h3_ref, w_ref, b_ref, o_ref, *, act, split):
    def cat(r):
        return jnp.concatenate([r[0], r[1]], axis=1)

    acc = jnp.dot(cat(x_ref), w_ref[0], preferred_element_type=_F32)
    acc = acc + jnp.dot(cat(h1_ref), w_ref[1], preferred_element_type=_F32)
    acc = acc + jnp.dot(cat(h2_ref), w_ref[2], preferred_element_type=_F32)
    acc = acc + jnp.dot(cat(h3_ref), w_ref[3], preferred_element_type=_F32)
    acc = acc + b_ref[...]
    if act:
        acc = jnp.where(acc > 0.0, acc, jnp.exp(jnp.minimum(acc, 0.0)) - 1.0)
    if split:
        o_ref[0] = acc[:, :HD]
        o_ref[1] = acc[:, HD:]
    else:
        o_ref[...] = acc


def _dense(x2, h1, h2, h3, w, b, act, split):
    bn = 512
    body = functools.partial(_tc_body, act=act, split=split)
    if split:
        out_shape = jax.ShapeDtypeStruct((2, NP, HD), _F32)
        out_spec = pl.BlockSpec((2, bn, HD), lambda i: (0, i, 0))
    else:
        out_shape = jax.ShapeDtypeStruct((NP, DD), _F32)
        out_spec = pl.BlockSpec((bn, DD), lambda i: (i, 0))
    return pl.pallas_call(
        body,
        grid=(NP // bn,),
        in_specs=[pl.BlockSpec((2, bn, HD), lambda i: (0, i, 0))] * 4
        + [pl.BlockSpec((4, DD, DD), lambda i: (0, 0, 0)),
           pl.BlockSpec((1, DD), lambda i: (0, 0))],
        out_specs=out_spec,
        out_shape=out_shape,
    )(x2, h1, h2, h3, w, b)


def kernel(x, edge_index, weight, W1, b1, W2, b2, W3, b3):
    row, col = edge_index[0], edge_index[1]
    pad_e = EP - EE
    x2 = jnp.pad(x, ((0, NP - NN), (0, 0))).reshape(NP, 2, HD).transpose(1, 0, 2)
    rowp = jnp.concatenate(
        [row, jnp.full((pad_e,), NP - 1, jnp.int32)]).reshape(16, G, GRP)
    colp = jnp.concatenate(
        [col, jnp.full((pad_e,), NP - 1, jnp.int32)]).reshape(16, G, GRP)
    ewp = jnp.concatenate(
        [weight, jnp.zeros((pad_e,), _F32)]).reshape(16, G, GRP)

    norm, h1, h2, h3 = _layer1_call(x2, rowp, colp, ewp)
    y = _dense(x2, h1, h2, h3, W1, b1.reshape(1, DD), act=True, split=True)
    h1, h2, h3 = _prop_call(y, rowp, colp, norm)
    y = _dense(y, h1, h2, h3, W2, b2.reshape(1, DD), act=True, split=True)
    h1, h2, h3 = _prop_call(y, rowp, colp, norm)
    w3p = jnp.pad(W3, ((0, 0), (0, 0), (0, DD - CC)))
    b3p = jnp.pad(b3, (0, DD - CC)).reshape(1, DD)
    y = _dense(y, h1, h2, h3, w3p, b3p, act=False, split=False)
    return y[:NN, :CC]


# speedup vs baseline: 4.6843x; 1.0890x over previous
import functools

import jax
import jax.numpy as jnp
from jax import lax
from jax.experimental import pallas as pl
from jax.experimental.pallas import tpu as pltpu
from jax.experimental.pallas import tpu_sc as plsc

NN = 10000
NP = 10240
RPT = NP // 16
EE = 320000
GRP = 128
CH = 16
NCH = 10
G = CH * NCH
ET = G * GRP
EP = 16 * ET
DD = 128
HD = 64
CC = 10


def _rsqrt16(v):
    i = lax.bitcast_convert_type(v, jnp.int32)
    i = jnp.int32(0x5F3759DF) - (i >> 1)
    y = lax.bitcast_convert_type(i, jnp.float32)
    for _ in range(3):
        y = y * (1.5 - 0.5 * v * y * y)
    return jnp.where(v > 0.0, y, jnp.float32(0.0))


def _fill_zeros(ref, nrows, width):
    z = jnp.zeros((16,), jnp.float32)

    def body(i, _):
        for q in range(width // 16):
            ref[i, pl.ds(q * 16, 16)] = z
        return _

    lax.fori_loop(0, nrows, body, None, unroll=4)


def _zero_slice(zbuf, buf, base):
    for i in range(RPT // GRP):
        pltpu.sync_copy(zbuf, buf.at[pl.ds(base + i * GRP, GRP)])


def _scale(buf, nrm_ref, g):
    def sub(u, _):
        nv = nrm_ref[g, pl.ds(u * 16, 16)]
        for j in range(16):
            w = nv[j]
            r = u * 16 + j
            for q in range(HD // 16):
                sl = pl.ds(q * 16, 16)
                buf[r, sl] = buf[r, sl] * w
        return _

    lax.fori_loop(0, GRP // 16, sub, None)


def _scale_scatter(buf, col_f, tg, nrm_ref, g, acc):
    _scale(buf, nrm_ref, g)
    pltpu.sync_copy(buf, acc.at[col_f.at[tg]], add=True)


def _hop_res(src_hbm, row_f, col_f, nrmh_s, nrm2, bufs, gsem, nsem, ssems,
             acc):
    pltpu.async_copy(nrmh_s.at[pl.ds(0, CH)], nrm2[0], nsem)
    pltpu.async_copy(src_hbm.at[row_f.at[0]], bufs[0], gsem)
    pltpu.async_copy(src_hbm.at[row_f.at[1]], bufs[1], gsem)

    def cpair(p, _):
        for nb in range(2):
            t = p * 2 + nb
            pltpu.make_async_copy(
                nrmh_s.at[pl.ds(0, CH)], nrm2[nb], nsem).wait()

            @pl.when(t + 1 < NCH)
            def _():
                pltpu.async_copy(
                    nrmh_s.at[pl.ds((t + 1) * CH, CH)], nrm2[1 - nb], nsem)

            def quad(h, _):
                for b in range(4):
                    g4 = h * 4 + b
                    g = t * CH + g4
                    b2 = (b + 2) % 4
                    pltpu.make_async_copy(
                        src_hbm.at[row_f.at[g]], bufs[b], gsem).wait()
                    _scale(bufs[b], nrm2[nb], g4)
                    pltpu.async_copy(bufs[b], acc.at[col_f.at[g]],
                                     ssems[b], add=True)

                    @pl.when(g + 2 < G)
                    def _():
                        @pl.when(g >= 2)
                        def _():
                            pltpu.make_async_copy(
                                src_hbm.at[row_f.at[g]], bufs[b2],
                                ssems[b2]).wait()

                        pltpu.async_copy(
                            src_hbm.at[row_f.at[g + 2]], bufs[b2], gsem)

                return _

            lax.fori_loop(0, CH // 4, quad, None)
        return _

    lax.fori_loop(0, NCH // 2, cpair, None)
    for b in range(4):
        pltpu.make_async_copy(
            src_hbm.at[row_f.at[0]], bufs[b], ssems[b]).wait()


def _hop_chk(src_hbm, row_f, col_f, nrmh_s, nrm_c, rows_v, rows_w, gsem, acc):
    bufs = (rows_v, rows_w)

    def chunk(t, _):
        pltpu.sync_copy(nrmh_s.at[pl.ds(t * CH, CH)], nrm_c)
        pltpu.async_copy(src_hbm.at[row_f.at[t * CH]], bufs[0], gsem)

        def pair(h, _):
            for b in range(2):
                g = h * 2 + b
                tg = t * CH + g
                pltpu.make_async_copy(
                    src_hbm.at[row_f.at[tg]], bufs[b], gsem).wait()

                @pl.when(g < CH - 1)
                def _():
                    pltpu.async_copy(
                        src_hbm.at[row_f.at[tg + 1]], bufs[1 - b], gsem)

                _scale_scatter(bufs[b], col_f, tg, nrm_c, g, acc)
            return _

        lax.fori_loop(0, CH // 2, pair, None)
        return _

    lax.fori_loop(0, NCH, chunk, None)


def _hop_epilogue(c, base, out, k, zbuf, acc):
    plsc.subcore_barrier()
    pltpu.sync_copy(acc.at[pl.ds(base, RPT)], out.at[c, pl.ds(base, RPT)])
    if k < 2:
        _zero_slice(zbuf, acc, base)
    plsc.subcore_barrier()


def _sc_layer1(x2, rowh, colh, ewh, normo, h1o, h2o, h3o,
               row_f, col_f, ew_c, nrm_c, dinv2, idx_v, rows_v, rows_w,
               zbuf, gsem, deg_s, acc):
    c = lax.axis_index("c")
    s = lax.axis_index("s")
    base = s * RPT
    ewh_s = ewh.at[s]

    pltpu.sync_copy(rowh.at[s], row_f)
    pltpu.sync_copy(colh.at[s], col_f)
    _fill_zeros(zbuf, GRP, HD)
    _fill_zeros(dinv2, NP // 16, 16)
    iota16 = lax.iota(jnp.int32, 16)

    def zi(i, _):
        idx_v[pl.ds(i * 16, 16)] = i * 16 + iota16
        return _

    lax.fori_loop(0, NP // 256, zi, None, unroll=4)

    @pl.when(s == 0)
    def _():
        pltpu.sync_copy(dinv2, deg_s)

    _zero_slice(zbuf, acc, base)
    plsc.subcore_barrier()

    def dchunk(t, _):
        pltpu.sync_copy(ewh_s.at[pl.ds(t * CH, CH)], ew_c)

        def dg(g, _):
            tg = t * CH + g

            def dsub(u, _):
                sl = pl.ds(u * 16, 16)
                cv = col_f[tg, sl]
                plsc.addupdate_scatter(
                    dinv2, [cv >> 4, cv & 15], ew_c[g, sl])
                return _

            lax.fori_loop(0, GRP // 16, dsub, None)
            return _

        lax.fori_loop(0, CH, dg, None)
        return _

    lax.fori_loop(0, NCH, dchunk, None)
    pltpu.sync_copy(dinv2, deg_s.at[idx_v], add=True)
    plsc.subcore_barrier()

    pltpu.sync_copy(deg_s, dinv2)

    def di(i, _):
        dinv2[i, :] = _rsqrt16(dinv2[i, :])
        return _

    lax.fori_loop(0, NP // 16, di, None)

    def nchunk(t, _):
        pltpu.sync_copy(ewh_s.at[pl.ds(t * CH, CH)], ew_c)

        def ng(g, _):
            tg = t * CH + g

            def nsub(q, _):
                sl = pl.ds(q * 16, 16)
                rv = row_f[tg, sl]
                cv = col_f[tg, sl]
                a = plsc.load_gather(dinv2, [rv >> 4, rv & 15])
                b = plsc.load_gather(dinv2, [cv >> 4, cv & 15])
                nrm_c[g, sl] = a * b * ew_c[g, sl]
                return _

            lax.fori_loop(0, GRP // 16, nsub, None)
            return _

        lax.fori_loop(0, CH, ng, None)

        @pl.when(c == 0)
        def _():
            pltpu.sync_copy(nrm_c, normo.at[s, pl.ds(t * CH, CH)])

        return _

    lax.fori_loop(0, NCH, nchunk, None)
    plsc.subcore_barrier()

    srcs = (x2, h1o, h2o)
    outs = (h1o, h2o, h3o)
    for k in range(3):
        _hop_chk(srcs[k].at[c], row_f, col_f, normo.at[s], nrm_c,
                 rows_v, rows_w, gsem, acc)
        _hop_epilogue(c, base, outs[k], k, zbuf, acc)


def _sc_prop(yp, rowh, colh, normh, h1o, h2o, h3o,
             row_f, col_f, nrm_a, nrm_b, b0, b1, b2, b3, zbuf,
             gsem, nsem, s0, s1, s2, s3, acc):
    c = lax.axis_index("c")
    s = lax.axis_index("s")
    base = s * RPT

    pltpu.sync_copy(rowh.at[s], row_f)
    pltpu.sync_copy(colh.at[s], col_f)
    _fill_zeros(zbuf, GRP, HD)
    _zero_slice(zbuf, acc, base)
    plsc.subcore_barrier()

    srcs = (yp, h1o, h2o)
    outs = (h1o, h2o, h3o)
    for k in range(3):
        _hop_res(srcs[k].at[c], row_f, col_f, normh.at[s], (nrm_a, nrm_b),
                 (b0, b1, b2, b3), gsem, nsem, (s0, s1, s2, s3), acc)
        _hop_epilogue(c, base, outs[k], k, zbuf, acc)


_MESH = plsc.VectorSubcoreMesh(core_axis_name="c", subcore_axis_name="s")

_F32 = jnp.float32
_HSHAPE = jax.ShapeDtypeStruct((2, NP, HD), _F32)

_SC_PARAMS = pltpu.CompilerParams(use_tc_tiling_on_sc=False,
                                  needs_layout_passes=False)

_layer1_call = pl.kernel(
    _sc_layer1,
    out_type=(jax.ShapeDtypeStruct((16, G, GRP), _F32), _HSHAPE, _HSHAPE, _HSHAPE),
    mesh=_MESH,
    compiler_params=_SC_PARAMS,
    scratch_types=[
        pltpu.VMEM((G, GRP), jnp.int32),
        pltpu.VMEM((G, GRP), jnp.int32),
        pltpu.VMEM((CH, GRP), _F32),
        pltpu.VMEM((CH, GRP), _F32),
        pltpu.VMEM((NP // 16, 16), _F32),
        pltpu.VMEM((NP // 16,), jnp.int32),
        pltpu.VMEM((GRP, HD), _F32),
        pltpu.VMEM((GRP, HD), _F32),
        pltpu.VMEM((GRP, HD), _F32),
        pltpu.SemaphoreType.DMA,
        pltpu.VMEM_SHARED((NP // 16, 16), _F32),
        pltpu.VMEM_SHARED((NP, HD), _F32),
    ],
)

_prop_call = pl.kernel(
    _sc_prop,
    out_type=(_HSHAPE, _HSHAPE, _HSHAPE),
    mesh=_MESH,
    compiler_params=_SC_PARAMS,
    scratch_types=[
        pltpu.VMEM((G, GRP), jnp.int32),
        pltpu.VMEM((G, GRP), jnp.int32),
        pltpu.VMEM((CH, GRP), _F32),
        pltpu.VMEM((CH, GRP), _F32),
        pltpu.VMEM((GRP, HD), _F32),
        pltpu.VMEM((GRP, HD), _F32),
        pltpu.VMEM((GRP, HD), _F32),
        pltpu.VMEM((GRP, HD), _F32),
        pltpu.VMEM((GRP, HD), _F32),
        pltpu.SemaphoreType.DMA,
        pltpu.SemaphoreType.DMA,
        pltpu.SemaphoreType.DMA,
        pltpu.SemaphoreType.DMA,
        pltpu.SemaphoreType.DMA,
        pltpu.SemaphoreType.DMA,
        pltpu.VMEM_SHARED((NP, HD), _F32),
    ],
)


def _tc_body(x_ref, h1_ref, h2_ref, h3_ref, w_ref, b_ref, o_ref, *, act, split):
    def cat(r):
        return jnp.concatenate([r[0], r[1]], axis=1)

    acc = jnp.dot(cat(x_ref), w_ref[0], preferred_element_type=_F32)
    acc = acc + jnp.dot(cat(h1_ref), w_ref[1], preferred_element_type=_F32)
    acc = acc + jnp.dot(cat(h2_ref), w_ref[2], preferred_element_type=_F32)
    acc = acc + jnp.dot(cat(h3_ref), w_ref[3], preferred_element_type=_F32)
    acc = acc + b_ref[...]
    if act:
        acc = jnp.where(acc > 0.0, acc, jnp.exp(jnp.minimum(acc, 0.0)) - 1.0)
    if split:
        o_ref[0] = acc[:, :HD]
        o_ref[1] = acc[:, HD:]
    else:
        o_ref[...] = acc


def _dense(x2, h1, h2, h3, w, b, act, split):
    bn = 512
    body = functools.partial(_tc_body, act=act, split=split)
    if split:
        out_shape = jax.ShapeDtypeStruct((2, NP, HD), _F32)
        out_spec = pl.BlockSpec((2, bn, HD), lambda i: (0, i, 0))
    else:
        out_shape = jax.ShapeDtypeStruct((NP, DD), _F32)
        out_spec = pl.BlockSpec((bn, DD), lambda i: (i, 0))
    return pl.pallas_call(
        body,
        grid=(NP // bn,),
        in_specs=[pl.BlockSpec((2, bn, HD), lambda i: (0, i, 0))] * 4
        + [pl.BlockSpec((4, DD, DD), lambda i: (0, 0, 0)),
           pl.BlockSpec((1, DD), lambda i: (0, 0))],
        out_specs=out_spec,
        out_shape=out_shape,
    )(x2, h1, h2, h3, w, b)


def kernel(x, edge_index, weight, W1, b1, W2, b2, W3, b3):
    row, col = edge_index[0], edge_index[1]
    pad_e = EP - EE
    x2 = jnp.pad(x, ((0, NP - NN), (0, 0))).reshape(NP, 2, HD).transpose(1, 0, 2)
    rowp = jnp.concatenate(
        [row, jnp.full((pad_e,), NP - 1, jnp.int32)]).reshape(16, G, GRP)
    colp = jnp.concatenate(
        [col, jnp.full((pad_e,), NP - 1, jnp.int32)]).reshape(16, G, GRP)
    ewp = jnp.concatenate(
        [weight, jnp.zeros((pad_e,), _F32)]).reshape(16, G, GRP)

    norm, h1, h2, h3 = _layer1_call(x2, rowp, colp, ewp)
    y = _dense(x2, h1, h2, h3, W1, b1.reshape(1, DD), act=True, split=True)
    h1, h2, h3 = _prop_call(y, rowp, colp, norm)
    y = _dense(y, h1, h2, h3, W2, b2.reshape(1, DD), act=True, split=True)
    h1, h2, h3 = _prop_call(y, rowp, colp, norm)
    w3p = jnp.pad(W3, ((0, 0), (0, 0), (0, DD - CC)))
    b3p = jnp.pad(b3, (0, DD - CC)).reshape(1, DD)
    y = _dense(y, h1, h2, h3, w3p, b3p, act=False, split=False)
    return y[:NN, :CC]
